# Initial kernel scaffold; baseline (speedup 1.0000x reference)
import jax
import jax.numpy as jnp
from jax.experimental import pallas as pl


def kernel(x, edge_index, W_self_0, W_neigh_0, b_0, W_self_1, W_neigh_1, b_1):
    raise NotImplementedError("write your pallas kernel here")


# speedup vs baseline: 4.3501x; 4.3501x over previous
import jax
import jax.numpy as jnp
from jax import lax
from jax.experimental import pallas as pl
from jax.experimental.pallas import tpu as pltpu
from jax.experimental.pallas import tpu_sc as plsc

N = 10000
D = 128
E = 320000
NC = 2
NS = 16
NW = NC * NS
EPT = E // NW
K = 80
NBLK = EPT // K
NP = 10240
CH = NP // NS
L = 16

_MESH = plsc.VectorSubcoreMesh(core_axis_name="c", subcore_axis_name="s")


def _zero_rows(rows_v, nrows):
    def zrow(i, c):
        for j in range(D // L):
            rows_v[i, pl.ds(j * L, L)] = jnp.zeros((L,), jnp.float32)
        return c
    lax.fori_loop(0, nrows, zrow, 0)


def _agg_kernel(table, srcs, dsts, acc_out, src_v, dst_v, rows_v, accum_sh,
                sem):
    cid = lax.axis_index("c")
    sid = lax.axis_index("s")
    wid = sid * NC + cid
    row0 = sid * CH

    _zero_rows(rows_v, K)
    for j in range(CH // K):
        pltpu.sync_copy(rows_v, accum_sh.at[pl.ds(row0 + j * K, K)])
    plsc.subcore_barrier()

    base0 = wid * EPT

    def eblock(i, c):
        base = base0 + i * K
        pltpu.sync_copy(srcs.at[pl.ds(base, K)], src_v)
        pltpu.sync_copy(dsts.at[pl.ds(base, K)], dst_v)
        pltpu.async_copy(table.at[src_v], rows_v, sem).wait()
        for j in range(K // L):
            idx = dst_v[pl.ds(j * L, L)]
            pltpu.async_copy(rows_v.at[pl.ds(j * L, L)],
                             accum_sh.at[idx], sem, add=True).wait()
        return c
    lax.fori_loop(0, NBLK, eblock, 0)
    plsc.subcore_barrier()

    for j in range(CH // K):
        pltpu.sync_copy(accum_sh.at[pl.ds(row0 + j * K, K)], rows_v)
        pltpu.sync_copy(rows_v, acc_out.at[cid, pl.ds(row0 + j * K, K)])


_agg = pl.kernel(
    _agg_kernel,
    out_type=jax.ShapeDtypeStruct((NC, NP, D), jnp.float32),
    mesh=_MESH,
    scratch_types=[
        pltpu.VMEM((K,), jnp.int32),
        pltpu.VMEM((K,), jnp.int32),
        pltpu.VMEM((K, D), jnp.float32),
        pltpu.VMEM_SHARED((NP, D), jnp.float32),
        pltpu.SemaphoreType.DMA,
    ],
)


def _deg_kernel(dsts, deg_out, dst_v, rows_v, ones_v, deg_sh, sem):
    cid = lax.axis_index("c")
    sid = lax.axis_index("s")
    wid = sid * NC + cid
    row0 = sid * CH

    _zero_rows(rows_v, K)
    for j in range(CH // K):
        pltpu.sync_copy(rows_v, deg_sh.at[pl.ds(row0 + j * K, K)])

    def orow(i, c):
        for j in range(D // L):
            ones_v[i, pl.ds(j * L, L)] = jnp.ones((L,), jnp.float32)
        return c
    lax.fori_loop(0, L, orow, 0)
    plsc.subcore_barrier()

    base0 = wid * EPT

    def eblock(i, c):
        base = base0 + i * K
        pltpu.sync_copy(dsts.at[pl.ds(base, K)], dst_v)
        for j in range(K // L):
            idx = dst_v[pl.ds(j * L, L)]
            pltpu.async_copy(ones_v, deg_sh.at[idx], sem, add=True).wait()
        return c
    lax.fori_loop(0, NBLK, eblock, 0)
    plsc.subcore_barrier()

    for j in range(CH // K):
        pltpu.sync_copy(deg_sh.at[pl.ds(row0 + j * K, K)], rows_v)
        pltpu.sync_copy(rows_v, deg_out.at[cid, pl.ds(row0 + j * K, K)])


_deg = pl.kernel(
    _deg_kernel,
    out_type=jax.ShapeDtypeStruct((NC, NP, D), jnp.float32),
    mesh=_MESH,
    scratch_types=[
        pltpu.VMEM((K,), jnp.int32),
        pltpu.VMEM((K, D), jnp.float32),
        pltpu.VMEM((L, D), jnp.float32),
        pltpu.VMEM_SHARED((NP, D), jnp.float32),
        pltpu.SemaphoreType.DMA,
    ],
)

BLK = 1000


def _fused_layer0(x, acc, deg_parts, w_self, w_neigh, b2d):
    def body(h_ref, a_ref, dp_ref, ws_ref, wn_ref, b_ref, o_ref, inv_ref):
        deg = dp_ref[0, :, 0:1] + dp_ref[1, :, 0:1]
        inv = 1.0 / jnp.maximum(deg, 1.0)
        agg = (a_ref[0] + a_ref[1]) * inv
        y = jnp.dot(h_ref[...], ws_ref[...], preferred_element_type=jnp.float32)
        y = y + jnp.dot(agg, wn_ref[...], preferred_element_type=jnp.float32)
        o_ref[...] = jnp.maximum(y + b_ref[...], 0.0)
        inv_ref[...] = inv

    return pl.pallas_call(
        body,
        grid=(N // BLK,),
        in_specs=[
            pl.BlockSpec((BLK, D), lambda i: (i, 0)),
            pl.BlockSpec((NC, BLK, D), lambda i: (0, i, 0)),
            pl.BlockSpec((NC, BLK, D), lambda i: (0, i, 0)),
            pl.BlockSpec((D, D), lambda i: (0, 0)),
            pl.BlockSpec((D, D), lambda i: (0, 0)),
            pl.BlockSpec((1, D), lambda i: (0, 0)),
        ],
        out_specs=[
            pl.BlockSpec((BLK, D), lambda i: (i, 0)),
            pl.BlockSpec((BLK, 1), lambda i: (i, 0)),
        ],
        out_shape=[
            jax.ShapeDtypeStruct((N, D), jnp.float32),
            jax.ShapeDtypeStruct((N, 1), jnp.float32),
        ],
    )(x, acc, deg_parts, w_self, w_neigh, b2d)


def _fused_layer1(h, acc, inv_deg, w_self, w_neigh, b2d):
    def body(h_ref, a_ref, d_ref, ws_ref, wn_ref, b_ref, o_ref):
        agg = (a_ref[0] + a_ref[1]) * d_ref[...]
        y = jnp.dot(h_ref[...], ws_ref[...], preferred_element_type=jnp.float32)
        y = y + jnp.dot(agg, wn_ref[...], preferred_element_type=jnp.float32)
        o_ref[...] = y + b_ref[...]

    return pl.pallas_call(
        body,
        grid=(N // BLK,),
        in_specs=[
            pl.BlockSpec((BLK, D), lambda i: (i, 0)),
            pl.BlockSpec((NC, BLK, D), lambda i: (0, i, 0)),
            pl.BlockSpec((BLK, 1), lambda i: (i, 0)),
            pl.BlockSpec((D, D), lambda i: (0, 0)),
            pl.BlockSpec((D, D), lambda i: (0, 0)),
            pl.BlockSpec((1, D), lambda i: (0, 0)),
        ],
        out_specs=pl.BlockSpec((BLK, D), lambda i: (i, 0)),
        out_shape=jax.ShapeDtypeStruct((N, D), jnp.float32),
    )(h, acc, inv_deg, w_self, w_neigh, b2d)


def kernel(x, edge_index, W_self_0, W_neigh_0, b_0, W_self_1, W_neigh_1, b_1):
    src = edge_index[0]
    dst = edge_index[1]
    acc0 = _agg(x, src, dst)
    deg_parts = _deg(dst)
    h, inv_deg = _fused_layer0(x, acc0, deg_parts, W_self_0, W_neigh_0,
                               b_0.reshape(1, D))
    acc1 = _agg(h, src, dst)
    out = _fused_layer1(h, acc1, inv_deg, W_self_1, W_neigh_1,
                        b_1.reshape(1, D))
    return out


# speedup vs baseline: 8.7901x; 2.0207x over previous
import jax
import jax.numpy as jnp
from jax import lax
from jax.experimental import pallas as pl
from jax.experimental.pallas import tpu as pltpu
from jax.experimental.pallas import tpu_sc as plsc

N = 10000
D = 128
E = 320000
NC = 2
NS = 16
NW = NC * NS
EPT = E // NW
K = 80
NBLK = EPT // K
NP = 10240
CH = NP // NS
L = 16

_MESH = plsc.VectorSubcoreMesh(core_axis_name="c", subcore_axis_name="s")


def _zero_rows(rows_v, nrows):
    def zrow(i, c):
        for j in range(D // L):
            rows_v[i, pl.ds(j * L, L)] = jnp.zeros((L,), jnp.float32)
        return c
    lax.fori_loop(0, nrows, zrow, 0)


def _agg_kernel(table, srcs, dsts, acc_out, src_all, dst_all, rows0, rows1,
                accum_sh, sem_g0, sem_g1, sem_s):
    cid = lax.axis_index("c")
    sid = lax.axis_index("s")
    wid = sid * NC + cid
    row0 = sid * CH

    _zero_rows(rows0, K)
    for j in range(CH // K):
        pltpu.sync_copy(rows0, accum_sh.at[pl.ds(row0 + j * K, K)])

    base0 = wid * EPT
    pltpu.sync_copy(srcs.at[pl.ds(base0, EPT)], src_all)
    pltpu.sync_copy(dsts.at[pl.ds(base0, EPT)], dst_all)
    plsc.subcore_barrier()

    def gather_start(i, rows_v, sem):
        pltpu.make_async_copy(
            table.at[src_all.at[pl.ds(i * K, K)]], rows_v, sem).start()

    def gather_wait(i, rows_v, sem):
        pltpu.make_async_copy(
            table.at[src_all.at[pl.ds(i * K, K)]], rows_v, sem).wait()

    def scatter_block(i, rows_v):
        descs = []
        for j in range(K // L):
            idx = dst_all[pl.ds(i * K + j * L, L)]
            d = pltpu.make_async_copy(rows_v.at[pl.ds(j * L, L)],
                                      accum_sh.at[idx], sem_s)
            d.start(add=True)
            descs.append(d)
        for d in descs:
            d.wait()

    gather_start(0, rows0, sem_g0)

    def pair(t, c):
        i0 = 2 * t
        gather_wait(i0, rows0, sem_g0)
        gather_start(i0 + 1, rows1, sem_g1)
        scatter_block(i0, rows0)
        gather_wait(i0 + 1, rows1, sem_g1)
        gather_start(i0 + 2, rows0, sem_g0)
        scatter_block(i0 + 1, rows1)
        return c
    lax.fori_loop(0, (NBLK - 1) // 2, pair, 0)
    gather_wait(NBLK - 1, rows0, sem_g0)
    scatter_block(NBLK - 1, rows0)
    plsc.subcore_barrier()

    for j in range(CH // K):
        pltpu.sync_copy(accum_sh.at[pl.ds(row0 + j * K, K)], rows0)
        pltpu.sync_copy(rows0, acc_out.at[cid, pl.ds(row0 + j * K, K)])


_agg = pl.kernel(
    _agg_kernel,
    out_type=jax.ShapeDtypeStruct((NC, NP, D), jnp.float32),
    mesh=_MESH,
    scratch_types=[
        pltpu.VMEM((EPT,), jnp.int32),
        pltpu.VMEM((EPT,), jnp.int32),
        pltpu.VMEM((K, D), jnp.float32),
        pltpu.VMEM((K, D), jnp.float32),
        pltpu.VMEM_SHARED((NP, D), jnp.float32),
        pltpu.SemaphoreType.DMA,
        pltpu.SemaphoreType.DMA,
        pltpu.SemaphoreType.DMA,
    ],
)


def _deg_kernel(dsts, deg_out, dst_all, rows_v, ones_v, deg_sh, sem):
    cid = lax.axis_index("c")
    sid = lax.axis_index("s")
    wid = sid * NC + cid
    row0 = sid * CH

    _zero_rows(rows_v, K)
    for j in range(CH // K):
        pltpu.sync_copy(rows_v, deg_sh.at[pl.ds(row0 + j * K, K)])

    def orow(i, c):
        for j in range(D // L):
            ones_v[i, pl.ds(j * L, L)] = jnp.ones((L,), jnp.float32)
        return c
    lax.fori_loop(0, L, orow, 0)
    base0 = wid * EPT
    pltpu.sync_copy(dsts.at[pl.ds(base0, EPT)], dst_all)
    plsc.subcore_barrier()

    def eblock(i, c):
        descs = []
        for j in range(K // L):
            idx = dst_all[pl.ds(i * K + j * L, L)]
            d = pltpu.make_async_copy(ones_v, deg_sh.at[idx], sem)
            d.start(add=True)
            descs.append(d)
        for d in descs:
            d.wait()
        return c
    lax.fori_loop(0, NBLK, eblock, 0)
    plsc.subcore_barrier()

    for j in range(CH // K):
        pltpu.sync_copy(deg_sh.at[pl.ds(row0 + j * K, K)], rows_v)
        pltpu.sync_copy(rows_v, deg_out.at[cid, pl.ds(row0 + j * K, K)])


_deg = pl.kernel(
    _deg_kernel,
    out_type=jax.ShapeDtypeStruct((NC, NP, D), jnp.float32),
    mesh=_MESH,
    scratch_types=[
        pltpu.VMEM((EPT,), jnp.int32),
        pltpu.VMEM((K, D), jnp.float32),
        pltpu.VMEM((L, D), jnp.float32),
        pltpu.VMEM_SHARED((NP, D), jnp.float32),
        pltpu.SemaphoreType.DMA,
    ],
)

BLK = 1000


def _fused_layer0(x, acc, deg_parts, w_self, w_neigh, b2d):
    def body(h_ref, a_ref, dp_ref, ws_ref, wn_ref, b_ref, o_ref, inv_ref):
        deg = dp_ref[0, :, 0:1] + dp_ref[1, :, 0:1]
        inv = 1.0 / jnp.maximum(deg, 1.0)
        agg = (a_ref[0] + a_ref[1]) * inv
        y = jnp.dot(h_ref[...], ws_ref[...], preferred_element_type=jnp.float32)
        y = y + jnp.dot(agg, wn_ref[...], preferred_element_type=jnp.float32)
        o_ref[...] = jnp.maximum(y + b_ref[...], 0.0)
        inv_ref[...] = inv

    return pl.pallas_call(
        body,
        grid=(N // BLK,),
        in_specs=[
            pl.BlockSpec((BLK, D), lambda i: (i, 0)),
            pl.BlockSpec((NC, BLK, D), lambda i: (0, i, 0)),
            pl.BlockSpec((NC, BLK, D), lambda i: (0, i, 0)),
            pl.BlockSpec((D, D), lambda i: (0, 0)),
            pl.BlockSpec((D, D), lambda i: (0, 0)),
            pl.BlockSpec((1, D), lambda i: (0, 0)),
        ],
        out_specs=[
            pl.BlockSpec((BLK, D), lambda i: (i, 0)),
            pl.BlockSpec((BLK, 1), lambda i: (i, 0)),
        ],
        out_shape=[
            jax.ShapeDtypeStruct((N, D), jnp.float32),
            jax.ShapeDtypeStruct((N, 1), jnp.float32),
        ],
    )(x, acc, deg_parts, w_self, w_neigh, b2d)


def _fused_layer1(h, acc, inv_deg, w_self, w_neigh, b2d):
    def body(h_ref, a_ref, d_ref, ws_ref, wn_ref, b_ref, o_ref):
        agg = (a_ref[0] + a_ref[1]) * d_ref[...]
        y = jnp.dot(h_ref[...], ws_ref[...], preferred_element_type=jnp.float32)
        y = y + jnp.dot(agg, wn_ref[...], preferred_element_type=jnp.float32)
        o_ref[...] = y + b_ref[...]

    return pl.pallas_call(
        body,
        grid=(N // BLK,),
        in_specs=[
            pl.BlockSpec((BLK, D), lambda i: (i, 0)),
            pl.BlockSpec((NC, BLK, D), lambda i: (0, i, 0)),
            pl.BlockSpec((BLK, 1), lambda i: (i, 0)),
            pl.BlockSpec((D, D), lambda i: (0, 0)),
            pl.BlockSpec((D, D), lambda i: (0, 0)),
            pl.BlockSpec((1, D), lambda i: (0, 0)),
        ],
        out_specs=pl.BlockSpec((BLK, D), lambda i: (i, 0)),
        out_shape=jax.ShapeDtypeStruct((N, D), jnp.float32),
    )(h, acc, inv_deg, w_self, w_neigh, b2d)


def kernel(x, edge_index, W_self_0, W_neigh_0, b_0, W_self_1, W_neigh_1, b_1):
    src = edge_index[0]
    dst = edge_index[1]
    acc0 = _agg(x, src, dst)
    deg_parts = _deg(dst)
    h, inv_deg = _fused_layer0(x, acc0, deg_parts, W_self_0, W_neigh_0,
                               b_0.reshape(1, D))
    acc1 = _agg(h, src, dst)
    out = _fused_layer1(h, acc1, inv_deg, W_self_1, W_neigh_1,
                        b_1.reshape(1, D))
    return out


# speedup vs baseline: 10.4391x; 1.1876x over previous
import jax
import jax.numpy as jnp
from jax import lax
from jax.experimental import pallas as pl
from jax.experimental.pallas import tpu as pltpu
from jax.experimental.pallas import tpu_sc as plsc

N = 10000
D = 128
E = 320000
NC = 2
NS = 16
NW = NC * NS
EPT = E // NW
K = 80
NBLK = EPT // K
NP = 10240
CH = NP // NS
L = 16

_MESH = plsc.VectorSubcoreMesh(core_axis_name="c", subcore_axis_name="s")


def _zero_rows(rows_v, nrows):
    def zrow(i, c):
        for j in range(D // L):
            rows_v[i, pl.ds(j * L, L)] = jnp.zeros((L,), jnp.float32)
        return c
    lax.fori_loop(0, nrows, zrow, 0)


def _agg_kernel(table, srcs, dsts, acc_out, src_all, dst_all, rows0, rows1,
                accum_sh, sem_g0, sem_g1, sem_s):
    cid = lax.axis_index("c")
    sid = lax.axis_index("s")
    wid = sid * NC + cid
    row0 = sid * CH

    _zero_rows(rows0, K)
    for j in range(CH // K):
        pltpu.sync_copy(rows0, accum_sh.at[pl.ds(row0 + j * K, K)])

    base0 = wid * EPT
    pltpu.sync_copy(srcs.at[pl.ds(base0, EPT)], src_all)
    pltpu.sync_copy(dsts.at[pl.ds(base0, EPT)], dst_all)
    plsc.subcore_barrier()

    def gather(i, rows_v, sem):
        return pltpu.make_async_copy(
            table.at[src_all.at[pl.ds(i * K, K)]], rows_v, sem)

    def scatter_rows(base, rows_v):
        descs = []
        for j in range(K // L):
            idx = dst_all[pl.ds(base + j * L, L)]
            d = pltpu.make_async_copy(rows_v.at[pl.ds(j * L, L)],
                                      accum_sh.at[idx], sem_s)
            d.start(add=True)
            descs.append(d)
        for d in descs:
            d.wait()

    gather(0, rows0, sem_g0).start()

    def pair(t, c):
        i0 = 2 * t
        gather(i0, rows0, sem_g0).wait()
        gather(i0 + 1, rows1, sem_g1).start()
        scatter_rows(i0 * K, rows0)
        gather(i0 + 2, rows0, sem_g0).start()
        gather(i0 + 1, rows1, sem_g1).wait()
        scatter_rows((i0 + 1) * K, rows1)
        return c
    lax.fori_loop(0, (NBLK - 1) // 2, pair, 0)
    gather(NBLK - 1, rows0, sem_g0).wait()
    scatter_rows((NBLK - 1) * K, rows0)
    plsc.subcore_barrier()

    for j in range(CH // K):
        pltpu.sync_copy(accum_sh.at[pl.ds(row0 + j * K, K)], rows0)
        pltpu.sync_copy(rows0, acc_out.at[cid, pl.ds(row0 + j * K, K)])


_agg = pl.kernel(
    _agg_kernel,
    out_type=jax.ShapeDtypeStruct((NC, NP, D), jnp.float32),
    mesh=_MESH,
    scratch_types=[
        pltpu.VMEM((EPT,), jnp.int32),
        pltpu.VMEM((EPT,), jnp.int32),
        pltpu.VMEM((K, D), jnp.float32),
        pltpu.VMEM((K, D), jnp.float32),
        pltpu.VMEM_SHARED((NP, D), jnp.float32),
        pltpu.SemaphoreType.DMA,
        pltpu.SemaphoreType.DMA,
        pltpu.SemaphoreType.DMA,
    ],
)


def _deg_kernel(dsts, deg_out, dst_all, rows_v, ones_v, deg_sh, sem):
    cid = lax.axis_index("c")
    sid = lax.axis_index("s")
    wid = sid * NC + cid
    row0 = sid * CH

    _zero_rows(rows_v, K)
    for j in range(CH // K):
        pltpu.sync_copy(rows_v, deg_sh.at[pl.ds(row0 + j * K, K)])

    def orow(i, c):
        for j in range(D // L):
            ones_v[i, pl.ds(j * L, L)] = jnp.ones((L,), jnp.float32)
        return c
    lax.fori_loop(0, L, orow, 0)
    base0 = wid * EPT
    pltpu.sync_copy(dsts.at[pl.ds(base0, EPT)], dst_all)
    plsc.subcore_barrier()

    def eblock(i, c):
        descs = []
        for j in range(K // L):
            idx = dst_all[pl.ds(i * K + j * L, L)]
            d = pltpu.make_async_copy(ones_v, deg_sh.at[idx], sem)
            d.start(add=True)
            descs.append(d)
        for d in descs:
            d.wait()
        return c
    lax.fori_loop(0, NBLK, eblock, 0)
    plsc.subcore_barrier()

    for j in range(CH // K):
        pltpu.sync_copy(deg_sh.at[pl.ds(row0 + j * K, K)], rows_v)
        pltpu.sync_copy(rows_v, deg_out.at[cid, pl.ds(row0 + j * K, K)])


_deg = pl.kernel(
    _deg_kernel,
    out_type=jax.ShapeDtypeStruct((NC, NP, D), jnp.float32),
    mesh=_MESH,
    scratch_types=[
        pltpu.VMEM((EPT,), jnp.int32),
        pltpu.VMEM((K, D), jnp.float32),
        pltpu.VMEM((L, D), jnp.float32),
        pltpu.VMEM_SHARED((NP, D), jnp.float32),
        pltpu.SemaphoreType.DMA,
    ],
)

BLK = 1000


def _fused_layer0(x, acc, deg_parts, w_self, w_neigh, b2d):
    def body(h_ref, a_ref, dp_ref, ws_ref, wn_ref, b_ref, o_ref, inv_ref):
        deg = dp_ref[0, :, 0:1] + dp_ref[1, :, 0:1]
        inv = 1.0 / jnp.maximum(deg, 1.0)
        agg = (a_ref[0] + a_ref[1]) * inv
        y = jnp.dot(h_ref[...], ws_ref[...], preferred_element_type=jnp.float32)
        y = y + jnp.dot(agg, wn_ref[...], preferred_element_type=jnp.float32)
        o_ref[...] = jnp.maximum(y + b_ref[...], 0.0)
        inv_ref[...] = inv

    return pl.pallas_call(
        body,
        grid=(N // BLK,),
        in_specs=[
            pl.BlockSpec((BLK, D), lambda i: (i, 0)),
            pl.BlockSpec((NC, BLK, D), lambda i: (0, i, 0)),
            pl.BlockSpec((NC, BLK, D), lambda i: (0, i, 0)),
            pl.BlockSpec((D, D), lambda i: (0, 0)),
            pl.BlockSpec((D, D), lambda i: (0, 0)),
            pl.BlockSpec((1, D), lambda i: (0, 0)),
        ],
        out_specs=[
            pl.BlockSpec((BLK, D), lambda i: (i, 0)),
            pl.BlockSpec((BLK, 1), lambda i: (i, 0)),
        ],
        out_shape=[
            jax.ShapeDtypeStruct((N, D), jnp.float32),
            jax.ShapeDtypeStruct((N, 1), jnp.float32),
        ],
    )(x, acc, deg_parts, w_self, w_neigh, b2d)


def _fused_layer1(h, acc, inv_deg, w_self, w_neigh, b2d):
    def body(h_ref, a_ref, d_ref, ws_ref, wn_ref, b_ref, o_ref):
        agg = (a_ref[0] + a_ref[1]) * d_ref[...]
        y = jnp.dot(h_ref[...], ws_ref[...], preferred_element_type=jnp.float32)
        y = y + jnp.dot(agg, wn_ref[...], preferred_element_type=jnp.float32)
        o_ref[...] = y + b_ref[...]

    return pl.pallas_call(
        body,
        grid=(N // BLK,),
        in_specs=[
            pl.BlockSpec((BLK, D), lambda i: (i, 0)),
            pl.BlockSpec((NC, BLK, D), lambda i: (0, i, 0)),
            pl.BlockSpec((BLK, 1), lambda i: (i, 0)),
            pl.BlockSpec((D, D), lambda i: (0, 0)),
            pl.BlockSpec((D, D), lambda i: (0, 0)),
            pl.BlockSpec((1, D), lambda i: (0, 0)),
        ],
        out_specs=pl.BlockSpec((BLK, D), lambda i: (i, 0)),
        out_shape=jax.ShapeDtypeStruct((N, D), jnp.float32),
    )(h, acc, inv_deg, w_self, w_neigh, b2d)


def kernel(x, edge_index, W_self_0, W_neigh_0, b_0, W_self_1, W_neigh_1, b_1):
    src = edge_index[0]
    dst = edge_index[1]
    acc0 = _agg(x, src, dst)
    deg_parts = _deg(dst)
    h, inv_deg = _fused_layer0(x, acc0, deg_parts, W_self_0, W_neigh_0,
                               b_0.reshape(1, D))
    acc1 = _agg(h, src, dst)
    out = _fused_layer1(h, acc1, inv_deg, W_self_1, W_neigh_1,
                        b_1.reshape(1, D))
    return out
